# Initial kernel scaffold; baseline (speedup 1.0000x reference)
import jax
import jax.numpy as jnp
from jax.experimental import pallas as pl


def kernel(input_tensor, emb_table, pos_table):
    raise NotImplementedError("write your pallas kernel here")


# speedup vs baseline: 5.6382x; 5.6382x over previous
import functools

import jax
import jax.numpy as jnp
from jax import lax
from jax.experimental import pallas as pl
from jax.experimental.pallas import tpu as pltpu
from jax.experimental.pallas import tpu_sc as plsc

B = 4096
S = 512
D = 32
NC = 2
NS = 16
NW = NC * NS
ROWS_PER_W = B // NW
IDX_CHUNK = 128
N_CHUNKS = S // IDX_CHUNK
LANES = 16


def _sc_body(in_hbm, pos_hbm, table_hbm, out_hbm, idx_v, buf, pos_v, sem, psem):
    wid = lax.axis_index("s") * NC + lax.axis_index("c")
    base = wid * ROWS_PER_W

    pltpu.async_copy(pos_hbm, pos_v, psem).wait()

    def batch_body(i, _):
        b = base + i
        pltpu.async_copy(in_hbm.at[b], idx_v, psem).wait()
        cps = []
        for j in range(N_CHUNKS):
            cp = pltpu.make_async_copy(
                table_hbm.at[idx_v.at[j]],
                buf.at[pl.ds(j * IDX_CHUNK, IDX_CHUNK)],
                sem,
            )
            cp.start()
            cps.append(cp)
        for cp in cps:
            cp.wait()

        def add_body(r, _):
            for k in range(D // LANES):
                sl = (r, pl.ds(k * LANES, LANES))
                plsc.addupdate(buf.at[sl], pos_v[sl])
            return 0

        lax.fori_loop(0, S, add_body, 0, unroll=4)
        pltpu.async_copy(buf, out_hbm.at[b], psem).wait()
        return 0

    lax.fori_loop(0, ROWS_PER_W, batch_body, 0)


@jax.jit
def _run(input_i32, emb_table, pos_block):
    mesh = plsc.VectorSubcoreMesh(
        core_axis_name="c", subcore_axis_name="s", num_cores=NC, num_subcores=NS
    )
    f = pl.kernel(
        _sc_body,
        out_type=jax.ShapeDtypeStruct((B, S, D), jnp.float32),
        mesh=mesh,
        scratch_types=[
            pltpu.VMEM((N_CHUNKS, IDX_CHUNK), jnp.int32),
            pltpu.VMEM((S, D), jnp.float32),
            pltpu.VMEM((S, D), jnp.float32),
            pltpu.SemaphoreType.DMA,
            pltpu.SemaphoreType.DMA,
        ],
        compiler_params=pltpu.CompilerParams(use_tc_tiling_on_sc=False),
    )
    return f(input_i32, pos_block, emb_table)


def kernel(input_tensor, emb_table, pos_table):
    input_i32 = input_tensor.astype(jnp.int32).reshape(B, N_CHUNKS, IDX_CHUNK)
    pos_block = pos_table[1 : S + 1]
    return _run(input_i32, emb_table, pos_block)
